# baseline (speedup 1.0000x reference)
import functools

import jax
import jax.numpy as jnp
from jax import lax
from jax.experimental import pallas as pl
from jax.experimental.pallas import tpu as pltpu
from jax.experimental.pallas import tpu_sc as plsc

N_EMBED = 1024
MAX_SEQ_LEN = 8192
BATCH = 4

SC_BATCH = 2

S_BLK = 1024
NUM_BLKS = MAX_SEQ_LEN // S_BLK


def _make_sc_broadcast(num_batches):
    info = plsc.get_sparse_core_info()
    num_cores, num_subcores = info.num_cores, info.num_subcores
    num_workers = num_cores * num_subcores
    rows_per_worker = MAX_SEQ_LEN // num_workers

    mesh = plsc.VectorSubcoreMesh(core_axis_name="c", subcore_axis_name="s")

    chunk_rows = 64
    num_chunks = rows_per_worker // chunk_rows

    @functools.partial(
        pl.kernel,
        mesh=mesh,
        out_type=jax.ShapeDtypeStruct((num_batches, MAX_SEQ_LEN, N_EMBED), jnp.float32),
        scratch_types=[pltpu.VMEM((chunk_rows, N_EMBED), jnp.float32)],
    )
    def broadcast_rows(table_hbm, out_hbm, buf):
        wid = lax.axis_index("s") * num_cores + lax.axis_index("c")
        base = wid * rows_per_worker

        def body(i, carry):
            row0 = base + i * chunk_rows
            pltpu.sync_copy(table_hbm.at[pl.ds(row0, chunk_rows)], buf)
            for b in range(num_batches):
                pltpu.sync_copy(buf, out_hbm.at[b, pl.ds(row0, chunk_rows)])
            return carry

        lax.fori_loop(0, num_chunks, body, 0)

    return broadcast_rows


_sc_broadcast = _make_sc_broadcast(SC_BATCH)


def _tc_copy_body(table_ref, out_ref):
    blk = table_ref[...]
    for b in range(BATCH - SC_BATCH):
        out_ref[b] = blk


def _tc_broadcast(pos_table):
    return pl.pallas_call(
        _tc_copy_body,
        grid=(NUM_BLKS,),
        in_specs=[
            pl.BlockSpec((S_BLK, N_EMBED), lambda i: (i, 0)),
        ],
        out_specs=pl.BlockSpec((BATCH - SC_BATCH, S_BLK, N_EMBED), lambda i: (0, i, 0)),
        out_shape=jax.ShapeDtypeStruct(
            (BATCH - SC_BATCH, MAX_SEQ_LEN, N_EMBED), jnp.float32
        ),
    )(pos_table)


@jax.jit
def _hybrid(pos_table):
    sc_out = _sc_broadcast(pos_table)
    tc_out = _tc_broadcast(pos_table)
    return jnp.concatenate([sc_out, tc_out], axis=0)


def kernel(input_ids, pos_table):
    del input_ids
    return _hybrid(pos_table)


# speedup vs baseline: 3.2965x; 3.2965x over previous
import jax
import jax.numpy as jnp
from jax.experimental import pallas as pl

N_EMBED = 1024
MAX_SEQ_LEN = 8192
BATCH = 4

S_BLK = 1024
NUM_BLKS = MAX_SEQ_LEN // S_BLK


def _copy_body(table_ref, out_ref):
    blk = table_ref[...]
    for b in range(BATCH):
        out_ref[b] = blk


@jax.jit
def _tc_broadcast(pos_table):
    return pl.pallas_call(
        _copy_body,
        grid=(NUM_BLKS,),
        in_specs=[
            pl.BlockSpec((S_BLK, N_EMBED), lambda i: (i, 0)),
        ],
        out_specs=pl.BlockSpec((BATCH, S_BLK, N_EMBED), lambda i: (0, i, 0)),
        out_shape=jax.ShapeDtypeStruct((BATCH, MAX_SEQ_LEN, N_EMBED), jnp.float32),
    )(pos_table)


def kernel(input_ids, pos_table):
    del input_ids
    return _tc_broadcast(pos_table)
